# baseline (speedup 1.0000x reference)
import functools

import jax
import jax.numpy as jnp
from jax import lax
from jax.experimental import pallas as pl
from jax.experimental.pallas import tpu as pltpu
from jax.experimental.pallas import tpu_sc as plsc

B, S, V, D, P = 4, 4096, 100000, 1024, 576

L = 16
NC, NS = 2, 16
NW = NC * NS
TPB = NW // B

K = 32
NB = 3
T01 = 1024
G01 = (T01 - P) // 2
NV = P // 2
GR = (S - T01) // (TPB - 2)
NGS = G01 // K
NVC = NV // K
NSCAN = 9


def _pipe2(nch, start, finish):
    h = [None] * nch
    w = [None] * nch
    for c in range(min(NB - 1, nch)):
        h[c] = start(c, c % NB)
    for c in range(nch):
        h[c].wait()
        w[c] = finish(c, c % NB)
        n = c + NB - 1
        if n < nch:
            if n - NB >= 0:
                w[n - NB].wait()
            h[n] = start(n, n % NB)
    for c in range(max(0, nch - NB), nch):
        w[c].wait()


def _body(weight_hbm, ids_hbm, vis_hbm, img_hbm, out_hbm,
          scan_v, img_v, ids_all, idx_all, didxg, didxv,
          rows_a, rows_b, rows_c,
          isem, gsem_a, gsem_b, gsem_c, wsem_a, wsem_b, wsem_c):
    rows = (rows_a, rows_b, rows_c)
    gsem = (gsem_a, gsem_b, gsem_c)
    wsem = (wsem_a, wsem_b, wsem_c)

    cid = lax.axis_index("c")
    sid = lax.axis_index("s")
    wid = cid * NS + sid
    b = wid // TPB
    j = wid - b * TPB
    base = b * S
    iota = lax.iota(jnp.int32, L)

    @pl.when(j >= 2)
    def _dense():
        start0 = base + T01 + (j - 2) * GR
        pltpu.sync_copy(ids_hbm.at[pl.ds(start0, GR)], ids_all)

        def start(c, a):
            return pltpu.async_copy(
                weight_hbm.at[ids_all.at[pl.ds(c * K, K)]], rows[a], gsem[a])

        def finish(c, a):
            return pltpu.async_copy(
                rows[a], out_hbm.at[pl.ds(start0 + c * K, K)], wsem[a])

        _pipe2(GR // K, start, finish)

    @pl.when(j < 2)
    def _span():
        pltpu.sync_copy(ids_hbm.at[pl.ds(base, NSCAN * L)], scan_v)
        pltpu.sync_copy(img_hbm, img_v)
        img = img_v[...]

        acc = jnp.full((L,), S, jnp.int32)
        for i in range(NSCAN):
            vals = scan_v[pl.ds(i * L, L)]
            acc = jnp.minimum(acc, jnp.where(vals == img, iota + i * L, S))
        for sft in (1, 2, 4, 8):
            rot = acc.at[(iota + sft) & (L - 1)].get(mode="promise_in_bounds")
            acc = jnp.minimum(acc, rot)
        f = acc

        r0 = j * G01
        for c in range(NGS):
            for u in range(K // L):
                r = iota + (r0 + c * K + u * L)
                q = jnp.where(r < f, r, r + P)
                didxg[c, pl.ds(u * L, L)] = base + q
        v0 = j * NV
        for c in range(NVC):
            for u in range(K // L):
                didxv[c, pl.ds(u * L, L)] = (
                    base + f + (v0 + c * K + u * L) + iota)
        hi = [pltpu.async_copy(ids_hbm.at[didxg.at[c]],
                               idx_all.at[pl.ds(c * K, K)], isem)
              for c in range(NGS)]
        for h in hi:
            h.wait()

        def startg(c, a):
            return pltpu.async_copy(
                weight_hbm.at[idx_all.at[pl.ds(c * K, K)]], rows[a], gsem[a])

        def finishg(c, a):
            return pltpu.async_copy(rows[a], out_hbm.at[didxg.at[c]], wsem[a])

        _pipe2(NGS, startg, finishg)

        def startv(c, a):
            return pltpu.async_copy(
                vis_hbm.at[pl.ds(b * P + v0 + c * K, K)], rows[a], gsem[a])

        def finishv(c, a):
            return pltpu.async_copy(rows[a], out_hbm.at[didxv.at[c]], wsem[a])

        _pipe2(NVC, startv, finishv)


_sc_call = functools.partial(
    pl.kernel,
    out_type=jax.ShapeDtypeStruct((B * S, D), jnp.float32),
    mesh=plsc.VectorSubcoreMesh(core_axis_name="c", subcore_axis_name="s"),
    scratch_types=[
        pltpu.VMEM((NSCAN * L,), jnp.int32),
        pltpu.VMEM((L,), jnp.int32),
        pltpu.VMEM((GR,), jnp.int32),
        pltpu.VMEM((G01,), jnp.int32),
        pltpu.VMEM((NGS, K), jnp.int32),
        pltpu.VMEM((NVC, K), jnp.int32),
        pltpu.VMEM((K, D), jnp.float32),
        pltpu.VMEM((K, D), jnp.float32),
        pltpu.VMEM((K, D), jnp.float32),
        pltpu.SemaphoreType.DMA,
        pltpu.SemaphoreType.DMA,
        pltpu.SemaphoreType.DMA,
        pltpu.SemaphoreType.DMA,
        pltpu.SemaphoreType.DMA,
        pltpu.SemaphoreType.DMA,
        pltpu.SemaphoreType.DMA,
    ],
)(_body)


def kernel(input_ids, weight, vision_features, image_token_id):
    ids = input_ids.reshape(B * S).astype(jnp.int32)
    vis = vision_features.reshape(B * P, D).astype(jnp.float32)
    img = jnp.full((L,), image_token_id, dtype=jnp.int32)
    out = _sc_call(weight.astype(jnp.float32), ids, vis, img)
    return out.reshape(B, S, D)


# speedup vs baseline: 1.0097x; 1.0097x over previous
import functools

import jax
import jax.numpy as jnp
from jax import lax
from jax.experimental import pallas as pl
from jax.experimental.pallas import tpu as pltpu
from jax.experimental.pallas import tpu_sc as plsc

B, S, V, D, P = 4, 4096, 100000, 1024, 576

L = 16
NC, NS = 2, 16
NW = NC * NS
TPB = NW // B

K = 16
NB = 6
T01 = 1024
G01 = (T01 - P) // 2
NV = P // 2
GR = (S - T01) // (TPB - 2)
NGS = G01 // K
NVC = NV // K
NSCAN = 9


def _pipe2(nch, start, finish):
    h = [None] * nch
    w = [None] * nch
    for c in range(min(NB - 1, nch)):
        h[c] = start(c, c % NB)
    for c in range(nch):
        h[c].wait()
        w[c] = finish(c, c % NB)
        n = c + NB - 1
        if n < nch:
            if n - NB >= 0:
                w[n - NB].wait()
            h[n] = start(n, n % NB)
    for c in range(max(0, nch - NB), nch):
        w[c].wait()


def _body(weight_hbm, ids_hbm, vis_hbm, img_hbm, out_hbm,
          scan_v, img_v, ids_all, idx_all, didxg, didxv,
          rows_a, rows_b, rows_c, rows_d, rows_e, rows_f,
          isem, gsem_a, gsem_b, gsem_c, gsem_d, gsem_e, gsem_f,
          wsem_a, wsem_b, wsem_c, wsem_d, wsem_e, wsem_f):
    rows = (rows_a, rows_b, rows_c, rows_d, rows_e, rows_f)
    gsem = (gsem_a, gsem_b, gsem_c, gsem_d, gsem_e, gsem_f)
    wsem = (wsem_a, wsem_b, wsem_c, wsem_d, wsem_e, wsem_f)

    cid = lax.axis_index("c")
    sid = lax.axis_index("s")
    wid = cid * NS + sid
    b = wid // TPB
    j = wid - b * TPB
    base = b * S
    iota = lax.iota(jnp.int32, L)

    @pl.when(j >= 2)
    def _dense():
        start0 = base + T01 + (j - 2) * GR
        pltpu.sync_copy(ids_hbm.at[pl.ds(start0, GR)], ids_all)

        def start(c, a):
            return pltpu.async_copy(
                weight_hbm.at[ids_all.at[pl.ds(c * K, K)]], rows[a], gsem[a])

        def finish(c, a):
            return pltpu.async_copy(
                rows[a], out_hbm.at[pl.ds(start0 + c * K, K)], wsem[a])

        _pipe2(GR // K, start, finish)

    @pl.when(j < 2)
    def _span():
        pltpu.sync_copy(ids_hbm.at[pl.ds(base, NSCAN * L)], scan_v)
        pltpu.sync_copy(img_hbm, img_v)
        img = img_v[...]

        acc = jnp.full((L,), S, jnp.int32)
        for i in range(NSCAN):
            vals = scan_v[pl.ds(i * L, L)]
            acc = jnp.minimum(acc, jnp.where(vals == img, iota + i * L, S))
        for sft in (1, 2, 4, 8):
            rot = acc.at[(iota + sft) & (L - 1)].get(mode="promise_in_bounds")
            acc = jnp.minimum(acc, rot)
        f = acc

        r0 = j * G01
        for c in range(NGS):
            for u in range(K // L):
                r = iota + (r0 + c * K + u * L)
                q = jnp.where(r < f, r, r + P)
                didxg[c, pl.ds(u * L, L)] = base + q
        v0 = j * NV
        for c in range(NVC):
            for u in range(K // L):
                didxv[c, pl.ds(u * L, L)] = (
                    base + f + (v0 + c * K + u * L) + iota)
        hi = [pltpu.async_copy(ids_hbm.at[didxg.at[c]],
                               idx_all.at[pl.ds(c * K, K)], isem)
              for c in range(NGS)]
        for h in hi:
            h.wait()

        def startg(c, a):
            return pltpu.async_copy(
                weight_hbm.at[idx_all.at[pl.ds(c * K, K)]], rows[a], gsem[a])

        def finishg(c, a):
            return pltpu.async_copy(rows[a], out_hbm.at[didxg.at[c]], wsem[a])

        _pipe2(NGS, startg, finishg)

        def startv(c, a):
            return pltpu.async_copy(
                vis_hbm.at[pl.ds(b * P + v0 + c * K, K)], rows[a], gsem[a])

        def finishv(c, a):
            return pltpu.async_copy(rows[a], out_hbm.at[didxv.at[c]], wsem[a])

        _pipe2(NVC, startv, finishv)


_sc_call = functools.partial(
    pl.kernel,
    out_type=jax.ShapeDtypeStruct((B * S, D), jnp.float32),
    mesh=plsc.VectorSubcoreMesh(core_axis_name="c", subcore_axis_name="s"),
    scratch_types=[
        pltpu.VMEM((NSCAN * L,), jnp.int32),
        pltpu.VMEM((L,), jnp.int32),
        pltpu.VMEM((GR,), jnp.int32),
        pltpu.VMEM((G01,), jnp.int32),
        pltpu.VMEM((NGS, K), jnp.int32),
        pltpu.VMEM((NVC, K), jnp.int32),
        pltpu.VMEM((K, D), jnp.float32),
        pltpu.VMEM((K, D), jnp.float32),
        pltpu.VMEM((K, D), jnp.float32),
        pltpu.VMEM((K, D), jnp.float32),
        pltpu.VMEM((K, D), jnp.float32),
        pltpu.VMEM((K, D), jnp.float32),
        pltpu.SemaphoreType.DMA,
        pltpu.SemaphoreType.DMA,
        pltpu.SemaphoreType.DMA,
        pltpu.SemaphoreType.DMA,
        pltpu.SemaphoreType.DMA,
        pltpu.SemaphoreType.DMA,
        pltpu.SemaphoreType.DMA,
        pltpu.SemaphoreType.DMA,
        pltpu.SemaphoreType.DMA,
        pltpu.SemaphoreType.DMA,
        pltpu.SemaphoreType.DMA,
        pltpu.SemaphoreType.DMA,
        pltpu.SemaphoreType.DMA,
    ],
)(_body)


def kernel(input_ids, weight, vision_features, image_token_id):
    ids = input_ids.reshape(B * S).astype(jnp.int32)
    vis = vision_features.reshape(B * P, D).astype(jnp.float32)
    img = jnp.full((L,), image_token_id, dtype=jnp.int32)
    out = _sc_call(weight.astype(jnp.float32), ids, vis, img)
    return out.reshape(B, S, D)
